# baseline (speedup 1.0000x reference)
import functools

import jax
import jax.numpy as jnp
from jax.experimental import pallas as pl

P = 32
V_BLOCK = 128


def _pointnet_block(pts_ref, cnt_ref, w1_ref, b1_ref, w2_ref, b2_ref,
                    w3_ref, b3_ref, out_ref):
    V = out_ref.shape[0]
    cnt = cnt_ref[...]
    pts = pts_ref[...]

    pts3 = pts.reshape(V, P, 3)
    p_iota = jax.lax.broadcasted_iota(jnp.int32, (V, P, 1), 1)
    empty = (cnt <= 0).reshape(V, 1, 1)
    pts3 = jnp.where(empty & (p_iota == 0), 0.0, pts3)

    x = pts3[:, :, 0:1].reshape(V * P, 1)
    y = pts3[:, :, 1:2].reshape(V * P, 1)
    z = pts3[:, :, 2:3].reshape(V * P, 1)
    w1 = w1_ref[...]
    h1 = x * w1[0:1, :] + y * w1[1:2, :] + z * w1[2:3, :] + b1_ref[...]
    h1 = jnp.maximum(h1, 0.0)

    h2 = jnp.dot(h1, w2_ref[...], preferred_element_type=jnp.float32)
    h2 = jnp.maximum(h2 + b2_ref[...], 0.0)
    h3 = jnp.dot(h2, w3_ref[...], preferred_element_type=jnp.float32)
    h3 = h3 + b3_ref[...]

    mask = p_iota < jnp.maximum(cnt, 1).reshape(V, 1, 1)
    masked = jnp.where(mask, h3.reshape(V, P, 256), jnp.float32(-1e30))
    out_ref[...] = jnp.max(masked, axis=1)


@functools.partial(jax.jit, static_argnames=())
def kernel(Frustum_Voxel, Frustum_Voxel_num, W1, b1, W2, b2, W3, b3):
    B, H, Wd, Pp, _ = Frustum_Voxel.shape
    N = B * H * Wd
    pts = Frustum_Voxel.reshape(N * Pp, 3)
    cnt = Frustum_Voxel_num.reshape(N, 1)
    nb = N // V_BLOCK

    feat = pl.pallas_call(
        _pointnet_block,
        grid=(nb,),
        in_specs=[
            pl.BlockSpec((V_BLOCK * P, 3), lambda i: (i, 0)),
            pl.BlockSpec((V_BLOCK, 1), lambda i: (i, 0)),
            pl.BlockSpec((3, 32), lambda i: (0, 0)),
            pl.BlockSpec((1, 32), lambda i: (0, 0)),
            pl.BlockSpec((32, 256), lambda i: (0, 0)),
            pl.BlockSpec((1, 256), lambda i: (0, 0)),
            pl.BlockSpec((256, 256), lambda i: (0, 0)),
            pl.BlockSpec((1, 256), lambda i: (0, 0)),
        ],
        out_specs=pl.BlockSpec((V_BLOCK, 256), lambda i: (i, 0)),
        out_shape=jax.ShapeDtypeStruct((N, 256), jnp.float32),
    )(pts, cnt, W1, b1.reshape(1, 32), W2, b2.reshape(1, 256),
      W3, b3.reshape(1, 256))

    return feat.reshape(B, H, Wd, 256)


# speedup vs baseline: 1.8492x; 1.8492x over previous
import jax
import jax.numpy as jnp
from jax import lax
from jax.experimental import pallas as pl

P = 32
V_BLOCK = 128


def _pointnet_block(pts_ref, cnt_ref, w1_ref, b1_ref, w2_ref, b2_ref,
                    w3_ref, b3_ref, out_ref):
    V = out_ref.shape[0]
    cnt = cnt_ref[...]
    pts_t = pts_ref[...]

    h1 = lax.dot_general(pts_t, w1_ref[...],
                         dimension_numbers=(((0,), (0,)), ((), ())),
                         preferred_element_type=jnp.float32)
    h1 = jnp.maximum(h1 + b1_ref[...], 0.0)

    h13 = h1.reshape(P, V, 32)
    empty32 = jnp.broadcast_to(cnt <= 0, (V, 32))
    slab0 = lax.broadcasted_iota(jnp.int32, (P, V, 32), 0) == 0
    h13 = jnp.where(slab0 & empty32[None], jnp.maximum(b1_ref[...], 0.0)[None],
                    h13)
    h1 = h13.reshape(P * V, 32)

    h2 = jnp.dot(h1, w2_ref[...], preferred_element_type=jnp.float32)
    h2 = jnp.maximum(h2 + b2_ref[...], 0.0)
    h3 = jnp.dot(h2, w3_ref[...], preferred_element_type=jnp.float32)
    h3 = h3.reshape(P, V, 256)

    cnt_b = jnp.broadcast_to(jnp.maximum(cnt, 1), (V, 256))
    mask = lax.broadcasted_iota(jnp.int32, (P, V, 256), 0) < cnt_b[None]
    masked = jnp.where(mask, h3, jnp.float32(-1e30))
    out_ref[...] = jnp.max(masked, axis=0) + b3_ref[...]


def kernel(Frustum_Voxel, Frustum_Voxel_num, W1, b1, W2, b2, W3, b3):
    B, H, Wd, Pp, _ = Frustum_Voxel.shape
    N = B * H * Wd
    nb = N // V_BLOCK

    t = Frustum_Voxel.reshape(nb, V_BLOCK, Pp, 3).transpose(0, 2, 1, 3)
    pts_t = t.reshape(nb * Pp * V_BLOCK, 3).T
    cnt = Frustum_Voxel_num.reshape(N, 1)

    feat = pl.pallas_call(
        _pointnet_block,
        grid=(nb,),
        in_specs=[
            pl.BlockSpec((3, Pp * V_BLOCK), lambda i: (0, i)),
            pl.BlockSpec((V_BLOCK, 1), lambda i: (i, 0)),
            pl.BlockSpec((3, 32), lambda i: (0, 0)),
            pl.BlockSpec((1, 32), lambda i: (0, 0)),
            pl.BlockSpec((32, 256), lambda i: (0, 0)),
            pl.BlockSpec((1, 256), lambda i: (0, 0)),
            pl.BlockSpec((256, 256), lambda i: (0, 0)),
            pl.BlockSpec((1, 256), lambda i: (0, 0)),
        ],
        out_specs=pl.BlockSpec((V_BLOCK, 256), lambda i: (i, 0)),
        out_shape=jax.ShapeDtypeStruct((N, 256), jnp.float32),
    )(pts_t, cnt, W1, b1.reshape(1, 32), W2, b2.reshape(1, 256),
      W3, b3.reshape(1, 256))

    return feat.reshape(B, H, Wd, 256)
